# baseline (speedup 1.0000x reference)
import jax
import jax.numpy as jnp
from jax.experimental import pallas as pl
from jax.experimental.pallas import tpu as pltpu

_B, _E, _A = 128, 4, 100000
_AC = 1024
_NC = 98
_AP = _NC * _AC
_ACW = 2048
_NW = _AP // _ACW
_G = _AC // 8
_IMAX = 2147483647


def _body(qt_ref, lp_ref, c_ref, idx_ref, qbuf, accM, accS, accI):
    g = pl.program_id(0)

    @pl.when(g == 0)
    def _init():
        accM[...] = jnp.full((8, _B), -jnp.inf, jnp.float32)
        accS[...] = jnp.zeros((8, _B), jnp.float32)
        accI[...] = jnp.full((8, _B), _IMAX, jnp.int32)

    @pl.when(g < _NC)
    def _scan():
        q = jnp.min(qt_ref[...], axis=0)
        qbuf[pl.ds(g * _AC, _AC), :] = q
        ids = (jax.lax.broadcasted_iota(jnp.int32, (_AC, _B), 0) + g * _AC)
        qv = jnp.where(ids < _A, q, -jnp.inf)
        q3 = qv.reshape(_G, 8, _B)
        i3 = ids.reshape(_G, 8, _B)
        m_c = jnp.max(q3, axis=0)
        i_c = jnp.min(jnp.where(q3 == m_c[None], i3, jnp.int32(_IMAX)),
                      axis=0)
        m_old = accM[...]
        m_run = jnp.maximum(m_old, m_c)
        s_c = jnp.sum(jnp.exp(q3 - m_run[None]), axis=0)
        accS[...] = accS[...] * jnp.exp(m_old - m_run) + s_c
        accI[...] = jnp.where(m_c > m_old, i_c, accI[...])
        accM[...] = m_run

    @pl.when(g == _NC - 1)
    def _fin():
        M, S, I = accM[...], accS[...], accI[...]
        m_g = jnp.max(M, axis=0, keepdims=True)
        lse = jnp.log(jnp.sum(S * jnp.exp(M - m_g), axis=0, keepdims=True))
        best = jnp.min(jnp.where(M == m_g, I, jnp.int32(_IMAX)),
                       axis=0, keepdims=True)
        c_ref[...] = jnp.broadcast_to(m_g + lse, (8, _B))
        idx_ref[...] = jnp.broadcast_to(best, (8, _B))

    @pl.when(g >= _NC)
    def _emit():
        j = g - _NC
        lp_ref[...] = qbuf[pl.ds(j * _ACW, _ACW), :] - c_ref[0:1, :]


def kernel(Qs):
    qt = jnp.transpose(Qs, (1, 2, 0))
    lp_t, c, idx = pl.pallas_call(
        _body,
        grid=(_NC + _NW,),
        in_specs=[pl.BlockSpec(
            (_E, _AC, _B),
            lambda g: (0, jnp.minimum(g, _NC - 1), 0))],
        out_specs=[
            pl.BlockSpec((_ACW, _B), lambda g: (jnp.maximum(g - _NC, 0), 0)),
            pl.BlockSpec((8, _B), lambda g: (0, 0)),
            pl.BlockSpec((8, _B), lambda g: (0, 0)),
        ],
        out_shape=[
            jax.ShapeDtypeStruct((_A, _B), jnp.float32),
            jax.ShapeDtypeStruct((8, _B), jnp.float32),
            jax.ShapeDtypeStruct((8, _B), jnp.int32),
        ],
        scratch_shapes=[
            pltpu.VMEM((_AP, _B), jnp.float32),
            pltpu.VMEM((8, _B), jnp.float32),
            pltpu.VMEM((8, _B), jnp.float32),
            pltpu.VMEM((8, _B), jnp.int32),
        ],
    )(qt)
    return jnp.transpose(lp_t), idx[0]
